# Initial kernel scaffold; baseline (speedup 1.0000x reference)
import jax
import jax.numpy as jnp
from jax.experimental import pallas as pl


def kernel(y, W0, W1, W2, W3):
    raise NotImplementedError("write your pallas kernel here")


# speedup vs baseline: 1.1103x; 1.1103x over previous
import functools

import jax
import jax.numpy as jnp
from jax import lax
from jax.experimental import pallas as pl
from jax.experimental.pallas import tpu as pltpu
from jax.experimental.pallas import tpu_sc as plsc

VOCAB = 100
D = 128
B = 16384
L = 16
NC = 2
NS = 16
NW = NC * NS
BPW = B // NW


def _body(yt_hbm, w0_hbm, w1_hbm, w2_hbm, w3_hbm, out_hbm,
          idx_v, t0, t1, t2, t3, acc):
    wid = lax.axis_index("s") * NC + lax.axis_index("c")
    base = wid * BPW

    for f in range(4):
        pltpu.sync_copy(yt_hbm.at[pl.ds(f * B + base, BPW)],
                        idx_v.at[pl.ds(f * BPW, BPW)])
    pltpu.sync_copy(w0_hbm, t0)
    pltpu.sync_copy(w1_hbm, t1)
    pltpu.sync_copy(w2_hbm, t2)
    pltpu.sync_copy(w3_hbm, t3)

    iota = lax.iota(jnp.int32, L)

    def group(g, carry):
        rv0 = idx_v[pl.ds(0 * BPW + g * L, L)] * D
        rv1 = idx_v[pl.ds(1 * BPW + g * L, L)] * D
        rv2 = idx_v[pl.ds(2 * BPW + g * L, L)] * D
        rv3 = idx_v[pl.ds(3 * BPW + g * L, L)] * D
        rowv = (iota + g * L) * D

        def col(c, carry2):
            colv = jnp.full((L,), c, dtype=jnp.int32)
            a = plsc.load_gather(t0, [rv0 + colv])
            a = a + plsc.load_gather(t1, [rv1 + colv])
            a = a + plsc.load_gather(t2, [rv2 + colv])
            a = a + plsc.load_gather(t3, [rv3 + colv])
            plsc.store_scatter(acc, [rowv + colv], a)
            return carry2

        lax.fori_loop(0, D, col, 0)
        return carry

    lax.fori_loop(0, BPW // L, group, 0)

    pltpu.sync_copy(acc, out_hbm.at[pl.ds(base * D, BPW * D)])


_emb = functools.partial(
    pl.kernel,
    mesh=plsc.VectorSubcoreMesh(core_axis_name="c", subcore_axis_name="s"),
    compiler_params=pltpu.CompilerParams(needs_layout_passes=False),
    out_type=jax.ShapeDtypeStruct((B * D,), jnp.float32),
    scratch_types=[
        pltpu.VMEM((4 * BPW,), jnp.int32),
        pltpu.VMEM((VOCAB * D,), jnp.float32),
        pltpu.VMEM((VOCAB * D,), jnp.float32),
        pltpu.VMEM((VOCAB * D,), jnp.float32),
        pltpu.VMEM((VOCAB * D,), jnp.float32),
        pltpu.VMEM((BPW * D,), jnp.float32),
    ],
)(_body)


@jax.jit
def kernel(y, W0, W1, W2, W3):
    yt = y.T.reshape(-1)
    out = _emb(yt, W0.reshape(-1), W1.reshape(-1), W2.reshape(-1),
               W3.reshape(-1))
    return out.reshape(B, D)


# speedup vs baseline: 1.6684x; 1.5027x over previous
import functools

import jax
import jax.numpy as jnp
from jax import lax
from jax.experimental import pallas as pl
from jax.experimental.pallas import tpu as pltpu
from jax.experimental.pallas import tpu_sc as plsc

VOCAB = 100
D = 128
B = 16384
L = 16
NC = 2
NS = 16
NW = NC * NS
BPW = B // NW


def _body(yt_hbm, w0_hbm, w1_hbm, w2_hbm, w3_hbm, out_hbm,
          idx_v, t0, t1, t2, t3, acc):
    wid = lax.axis_index("s") * NC + lax.axis_index("c")
    base = wid * BPW

    for f in range(4):
        pltpu.sync_copy(yt_hbm.at[pl.ds(f * B + base, BPW)],
                        idx_v.at[pl.ds(f * BPW, BPW)])
    pltpu.sync_copy(w0_hbm, t0)
    pltpu.sync_copy(w1_hbm, t1)
    pltpu.sync_copy(w2_hbm, t2)
    pltpu.sync_copy(w3_hbm, t3)

    iota = lax.iota(jnp.int32, L)

    def group(g, carry):
        rv0 = idx_v[pl.ds(0 * BPW + g * L, L)] * D
        rv1 = idx_v[pl.ds(1 * BPW + g * L, L)] * D
        rv2 = idx_v[pl.ds(2 * BPW + g * L, L)] * D
        rv3 = idx_v[pl.ds(3 * BPW + g * L, L)] * D
        rowv = (iota + g * L) * D

        @plsc.parallel_loop(0, D, unroll=8)
        def col(c):
            colv = jnp.full((L,), c, dtype=jnp.int32)
            a = plsc.load_gather(t0, [rv0 + colv])
            a = a + plsc.load_gather(t1, [rv1 + colv])
            a = a + plsc.load_gather(t2, [rv2 + colv])
            a = a + plsc.load_gather(t3, [rv3 + colv])
            plsc.store_scatter(acc, [rowv + colv], a)

        return carry

    lax.fori_loop(0, BPW // L, group, 0)

    pltpu.sync_copy(acc, out_hbm.at[pl.ds(base * D, BPW * D)])


_emb = functools.partial(
    pl.kernel,
    mesh=plsc.VectorSubcoreMesh(core_axis_name="c", subcore_axis_name="s"),
    compiler_params=pltpu.CompilerParams(needs_layout_passes=False),
    out_type=jax.ShapeDtypeStruct((B * D,), jnp.float32),
    scratch_types=[
        pltpu.VMEM((4 * BPW,), jnp.int32),
        pltpu.VMEM((VOCAB * D,), jnp.float32),
        pltpu.VMEM((VOCAB * D,), jnp.float32),
        pltpu.VMEM((VOCAB * D,), jnp.float32),
        pltpu.VMEM((VOCAB * D,), jnp.float32),
        pltpu.VMEM((BPW * D,), jnp.float32),
    ],
)(_body)


@jax.jit
def kernel(y, W0, W1, W2, W3):
    yt = y.T.reshape(-1)
    out = _emb(yt, W0.reshape(-1), W1.reshape(-1), W2.reshape(-1),
               W3.reshape(-1))
    return out.reshape(B, D)


# speedup vs baseline: 3.8170x; 2.2878x over previous
import functools

import jax
import jax.numpy as jnp
from jax import lax
from jax.experimental import pallas as pl
from jax.experimental.pallas import tpu as pltpu
from jax.experimental.pallas import tpu_sc as plsc

VOCAB = 100
D = 128
B = 16384
L = 16
NC = 2
NS = 16
NW = NC * NS
BPW = B // NW


def _body(yt_hbm, w0_hbm, w1_hbm, w2_hbm, w3_hbm, out_hbm,
          idx_v, t0, t1, t2, t3, acc):
    wid = lax.axis_index("s") * NC + lax.axis_index("c")
    base = wid * BPW

    for f in range(4):
        pltpu.sync_copy(yt_hbm.at[pl.ds(f * B + base, BPW)],
                        idx_v.at[pl.ds(f * BPW, BPW)])
    pltpu.sync_copy(w0_hbm, t0)
    pltpu.sync_copy(w1_hbm, t1)
    pltpu.sync_copy(w2_hbm, t2)
    pltpu.sync_copy(w3_hbm, t3)

    @plsc.parallel_loop(0, BPW // L, unroll=1)
    def group(g):
        rv0 = idx_v[pl.ds(0 * BPW + g * L, L)] * D
        rv1 = idx_v[pl.ds(1 * BPW + g * L, L)] * D
        rv2 = idx_v[pl.ds(2 * BPW + g * L, L)] * D
        rv3 = idx_v[pl.ds(3 * BPW + g * L, L)] * D
        for j in range(L):
            s0 = rv0[j]
            s1 = rv1[j]
            s2 = rv2[j]
            s3 = rv3[j]
            row = (g * L + j) * D
            for k in range(D // L):
                off = k * L
                a = t0[pl.ds(s0 + off, L)]
                a = a + t1[pl.ds(s1 + off, L)]
                a = a + t2[pl.ds(s2 + off, L)]
                a = a + t3[pl.ds(s3 + off, L)]
                acc[pl.ds(row + off, L)] = a

    pltpu.sync_copy(acc, out_hbm.at[pl.ds(base * D, BPW * D)])


_emb = functools.partial(
    pl.kernel,
    mesh=plsc.VectorSubcoreMesh(core_axis_name="c", subcore_axis_name="s"),
    compiler_params=pltpu.CompilerParams(needs_layout_passes=False),
    out_type=jax.ShapeDtypeStruct((B * D,), jnp.float32),
    scratch_types=[
        pltpu.VMEM((4 * BPW,), jnp.int32),
        pltpu.VMEM((VOCAB * D,), jnp.float32),
        pltpu.VMEM((VOCAB * D,), jnp.float32),
        pltpu.VMEM((VOCAB * D,), jnp.float32),
        pltpu.VMEM((VOCAB * D,), jnp.float32),
        pltpu.VMEM((BPW * D,), jnp.float32),
    ],
)(_body)


@jax.jit
def kernel(y, W0, W1, W2, W3):
    yt = y.T.reshape(-1)
    out = _emb(yt, W0.reshape(-1), W1.reshape(-1), W2.reshape(-1),
               W3.reshape(-1))
    return out.reshape(B, D)
